# baseline (speedup 1.0000x reference)
import jax
import jax.numpy as jnp
from jax.experimental import pallas as pl
from jax.experimental.pallas import tpu as pltpu

_MIN_MASK_AREA = 120000
_THRESHOLD = 0.5
_N, _H, _W = 128, 512, 512
_G = 8
_NP = _N // _G
_RING = 4
_LOOK = 2


def _body(masks_ref, labels_ref, boxes_ref, ids_ref,
          masks_out_ref, labels_out_ref, boxes_out_ref, ids_out_ref,
          ring_ref, cnt_ref, flag_ref, in_sems, out_sems):
    j = pl.program_id(0)

    def _in_copy(p):
        s = p % _RING
        return pltpu.make_async_copy(
            masks_ref.at[pl.ds(p * _G, _G)],
            ring_ref.at[pl.ds(s * _G, _G)], in_sems.at[s])

    def _out_drain(s):
        @pl.when(flag_ref[s] == _G)
        def _():
            pltpu.make_async_copy(
                ring_ref.at[pl.ds(s * _G, _G)],
                masks_out_ref.at[pl.ds(0, _G)], out_sems.at[s]).wait()

        @pl.when(flag_ref[s] < _G)
        def _():
            def _w(u, carry):
                pltpu.make_async_copy(
                    ring_ref.at[pl.ds(s * _G, 1)],
                    masks_out_ref.at[pl.ds(0, 1)], out_sems.at[s]).wait()
                return carry

            jax.lax.fori_loop(0, flag_ref[s], _w, 0)

        flag_ref[s] = 0

    @pl.when(j == 0)
    def _():
        cnt_ref[0] = 0
        for s in range(_RING):
            flag_ref[s] = 0
        for p in range(_LOOK):
            _in_copy(p).start()

    @pl.when(j + _LOOK < _NP)
    def _():
        s = (j + _LOOK) % _RING

        @pl.when(flag_ref[s] > 0)
        def _():
            _out_drain(s)

        _in_copy(j + _LOOK).start()

    _in_copy(j).wait()
    s_j = j % _RING
    blk = ring_ref[pl.ds(s_j * _G, _G)]
    sel = (blk.reshape(_G * _H, _W) > _THRESHOLD).astype(jnp.float32)
    row = jax.lax.broadcasted_iota(jnp.int32, (8, _G * _H), 0)
    col = jax.lax.broadcasted_iota(jnp.int32, (8, _G * _H), 1)
    lhs = ((col // _H) == row).astype(jnp.float32)
    acc = jax.lax.dot_general(
        lhs, sel, (((1,), (0,)), ((), ())),
        preferred_element_type=jnp.float32)
    passes = [jnp.sum(acc[k]) >= jnp.float32(_MIN_MASK_AREA)
              for k in range(_G)]
    all_pass = passes[0]
    for _k in range(1, _G):
        all_pass = all_pass & passes[_k]

    def _small_out(c, t):
        labels_out_ref[c] = labels_ref[t]
        ids_out_ref[c] = ids_ref[t]
        for colk in range(9):
            boxes_out_ref[c, colk] = boxes_ref[t, colk]

    @pl.when(all_pass)
    def _():
        c = cnt_ref[0]
        for k in range(_G):
            _small_out(c + k, _G * j + k)
        pltpu.make_async_copy(
            ring_ref.at[pl.ds(s_j * _G, _G)],
            masks_out_ref.at[pl.ds(c, _G)], out_sems.at[s_j]).start()
        flag_ref[s_j] = _G
        cnt_ref[0] = c + _G

    @pl.when(jnp.logical_not(all_pass))
    def _():
        for k in range(_G):
            @pl.when(passes[k])
            def _(k=k):
                c = cnt_ref[0]
                _small_out(c, _G * j + k)
                pltpu.make_async_copy(
                    ring_ref.at[pl.ds(s_j * _G + k, 1)],
                    masks_out_ref.at[pl.ds(c, 1)], out_sems.at[s_j]).start()
                flag_ref[s_j] = flag_ref[s_j] + 1
                cnt_ref[0] = c + 1

    @pl.when(j == _NP - 1)
    def _():
        def _drain(s, carry):
            @pl.when(flag_ref[s] > 0)
            def _():
                _out_drain(s)
            return carry

        jax.lax.fori_loop(0, _RING, _drain, 0)

        c = cnt_ref[0]

        def _fill(k, carry):
            @pl.when(k >= c)
            def _():
                _small_out(k, 0)
                cp = pltpu.make_async_copy(
                    masks_ref.at[pl.ds(0, 1)],
                    masks_out_ref.at[pl.ds(k, 1)], out_sems.at[0])
                cp.start()
                cp.wait()
            return carry

        jax.lax.fori_loop(0, _N, _fill, 0)


def kernel(masks, labels, boxes_3d, instance_ids):
    out_shape = (
        jax.ShapeDtypeStruct((_N, _H, _W), jnp.float32),
        jax.ShapeDtypeStruct((_N,), jnp.int32),
        jax.ShapeDtypeStruct((_N, 9), jnp.float32),
        jax.ShapeDtypeStruct((_N,), jnp.int32),
    )
    return pl.pallas_call(
        _body,
        grid=(_NP,),
        in_specs=[
            pl.BlockSpec(memory_space=pltpu.HBM),
            pl.BlockSpec(memory_space=pltpu.SMEM),
            pl.BlockSpec(memory_space=pltpu.SMEM),
            pl.BlockSpec(memory_space=pltpu.SMEM),
        ],
        out_specs=(
            pl.BlockSpec(memory_space=pltpu.HBM),
            pl.BlockSpec(memory_space=pltpu.SMEM),
            pl.BlockSpec(memory_space=pltpu.SMEM),
            pl.BlockSpec(memory_space=pltpu.SMEM),
        ),
        out_shape=out_shape,
        scratch_shapes=[
            pltpu.VMEM((_RING * _G, _H, _W), jnp.float32),
            pltpu.SMEM((1,), jnp.int32),
            pltpu.SMEM((_RING,), jnp.int32),
            pltpu.SemaphoreType.DMA((_RING,)),
            pltpu.SemaphoreType.DMA((_RING,)),
        ],
    )(masks, labels, boxes_3d, instance_ids)


# speedup vs baseline: 1.0229x; 1.0229x over previous
import jax
import jax.numpy as jnp
from jax.experimental import pallas as pl
from jax.experimental.pallas import tpu as pltpu

_MIN_MASK_AREA = 120000
_THRESHOLD = 0.5
_N, _H, _W = 128, 512, 512
_G = 16
_NP = _N // _G
_RING = 3
_LOOK = 2


def _body(masks_ref, labels_ref, boxes_ref, ids_ref,
          masks_out_ref, labels_out_ref, boxes_out_ref, ids_out_ref,
          ring_ref, cnt_ref, flag_ref, in_sems, out_sems):
    j = pl.program_id(0)

    def _in_copy(p):
        s = p % _RING
        return pltpu.make_async_copy(
            masks_ref.at[pl.ds(p * _G, _G)],
            ring_ref.at[pl.ds(s * _G, _G)], in_sems.at[s])

    def _out_drain(s):
        @pl.when(flag_ref[s] == _G)
        def _():
            pltpu.make_async_copy(
                ring_ref.at[pl.ds(s * _G, _G)],
                masks_out_ref.at[pl.ds(0, _G)], out_sems.at[s]).wait()

        @pl.when(flag_ref[s] < _G)
        def _():
            def _w(u, carry):
                pltpu.make_async_copy(
                    ring_ref.at[pl.ds(s * _G, 1)],
                    masks_out_ref.at[pl.ds(0, 1)], out_sems.at[s]).wait()
                return carry

            jax.lax.fori_loop(0, flag_ref[s], _w, 0)

        flag_ref[s] = 0

    @pl.when(j == 0)
    def _():
        cnt_ref[0] = 0
        for s in range(_RING):
            flag_ref[s] = 0
        for p in range(_LOOK):
            _in_copy(p).start()

    @pl.when(j + _LOOK < _NP)
    def _():
        s = (j + _LOOK) % _RING

        @pl.when(flag_ref[s] > 0)
        def _():
            _out_drain(s)

        _in_copy(j + _LOOK).start()

    _in_copy(j).wait()
    s_j = j % _RING
    blk = ring_ref[pl.ds(s_j * _G, _G)]
    sel = (blk.reshape(_G * _H, _W) > _THRESHOLD).astype(jnp.float32)
    _M = max(8, _G)
    row = jax.lax.broadcasted_iota(jnp.int32, (_M, _G * _H), 0)
    col = jax.lax.broadcasted_iota(jnp.int32, (_M, _G * _H), 1)
    lhs = ((col // _H) == row).astype(jnp.float32)
    acc = jax.lax.dot_general(
        lhs, sel, (((1,), (0,)), ((), ())),
        preferred_element_type=jnp.float32)
    passes = [jnp.sum(acc[k]) >= jnp.float32(_MIN_MASK_AREA)
              for k in range(_G)]
    all_pass = passes[0]
    for _k in range(1, _G):
        all_pass = all_pass & passes[_k]

    def _small_out(c, t):
        labels_out_ref[c] = labels_ref[t]
        ids_out_ref[c] = ids_ref[t]
        for colk in range(9):
            boxes_out_ref[c, colk] = boxes_ref[t, colk]

    @pl.when(all_pass)
    def _():
        c = cnt_ref[0]
        for k in range(_G):
            _small_out(c + k, _G * j + k)
        pltpu.make_async_copy(
            ring_ref.at[pl.ds(s_j * _G, _G)],
            masks_out_ref.at[pl.ds(c, _G)], out_sems.at[s_j]).start()
        flag_ref[s_j] = _G
        cnt_ref[0] = c + _G

    @pl.when(jnp.logical_not(all_pass))
    def _():
        for k in range(_G):
            @pl.when(passes[k])
            def _(k=k):
                c = cnt_ref[0]
                _small_out(c, _G * j + k)
                pltpu.make_async_copy(
                    ring_ref.at[pl.ds(s_j * _G + k, 1)],
                    masks_out_ref.at[pl.ds(c, 1)], out_sems.at[s_j]).start()
                flag_ref[s_j] = flag_ref[s_j] + 1
                cnt_ref[0] = c + 1

    @pl.when(j == _NP - 1)
    def _():
        def _drain(s, carry):
            @pl.when(flag_ref[s] > 0)
            def _():
                _out_drain(s)
            return carry

        jax.lax.fori_loop(0, _RING, _drain, 0)

        c = cnt_ref[0]

        def _fill(k, carry):
            @pl.when(k >= c)
            def _():
                _small_out(k, 0)
                cp = pltpu.make_async_copy(
                    masks_ref.at[pl.ds(0, 1)],
                    masks_out_ref.at[pl.ds(k, 1)], out_sems.at[0])
                cp.start()
                cp.wait()
            return carry

        jax.lax.fori_loop(0, _N, _fill, 0)


def kernel(masks, labels, boxes_3d, instance_ids):
    out_shape = (
        jax.ShapeDtypeStruct((_N, _H, _W), jnp.float32),
        jax.ShapeDtypeStruct((_N,), jnp.int32),
        jax.ShapeDtypeStruct((_N, 9), jnp.float32),
        jax.ShapeDtypeStruct((_N,), jnp.int32),
    )
    return pl.pallas_call(
        _body,
        grid=(_NP,),
        in_specs=[
            pl.BlockSpec(memory_space=pltpu.HBM),
            pl.BlockSpec(memory_space=pltpu.SMEM),
            pl.BlockSpec(memory_space=pltpu.SMEM),
            pl.BlockSpec(memory_space=pltpu.SMEM),
        ],
        out_specs=(
            pl.BlockSpec(memory_space=pltpu.HBM),
            pl.BlockSpec(memory_space=pltpu.SMEM),
            pl.BlockSpec(memory_space=pltpu.SMEM),
            pl.BlockSpec(memory_space=pltpu.SMEM),
        ),
        out_shape=out_shape,
        scratch_shapes=[
            pltpu.VMEM((_RING * _G, _H, _W), jnp.float32),
            pltpu.SMEM((1,), jnp.int32),
            pltpu.SMEM((_RING,), jnp.int32),
            pltpu.SemaphoreType.DMA((_RING,)),
            pltpu.SemaphoreType.DMA((_RING,)),
        ],
    )(masks, labels, boxes_3d, instance_ids)
